# baseline (speedup 1.0000x reference)
import functools

import jax
import jax.numpy as jnp
from jax import lax
from jax.experimental import pallas as pl
from jax.experimental.pallas import tpu as pltpu
from jax.experimental.pallas import tpu_sc as plsc

MASK_TOKEN_ID = 99999
NATIVE_DRAFT_LEN = 8

_info = plsc.get_sparse_core_info()
_NC = _info.num_cores
_NS = _info.num_subcores
_NW = _NC * _NS


@functools.partial(jax.jit, static_argnums=(1, 2))
def _gather_call(idx2d, hidden, b_per_w, table):
    nw, k = idx2d.shape
    rows = nw * b_per_w
    mesh = plsc.VectorSubcoreMesh(core_axis_name="c", subcore_axis_name="s")

    @functools.partial(
        pl.kernel,
        mesh=mesh,
        out_type=jax.ShapeDtypeStruct((rows, hidden), jnp.float32),
        scratch_types=[
            pltpu.VMEM((k,), jnp.int32),
            pltpu.VMEM((k, hidden), jnp.float32),
            pltpu.SemaphoreType.DMA,
        ],
        compiler_params=pltpu.CompilerParams(use_tc_tiling_on_sc=False),
    )
    def body(idx_hbm, table_hbm, out_hbm, idx_v, rows_v, sem):
        wid = lax.axis_index("s") * _NC + lax.axis_index("c")
        pltpu.sync_copy(idx_hbm.at[wid], idx_v)
        pltpu.async_copy(table_hbm.at[idx_v], rows_v, sem).wait()
        pltpu.sync_copy(
            rows_v.at[pl.ds(0, b_per_w)],
            out_hbm.at[pl.ds(wid * b_per_w, b_per_w)],
        )

    return body(idx2d, table)


def kernel(x, emb_table):
    bsz, seqlen = x.shape
    vocab, hidden = emb_table.shape
    t = seqlen + NATIVE_DRAFT_LEN - 1
    rows = bsz * t
    assert rows % _NW == 0
    b_per_w = rows // _NW
    k_pad = (-b_per_w) % 8
    k = b_per_w + k_pad

    mask = jnp.full((bsz, NATIVE_DRAFT_LEN - 1), MASK_TOKEN_ID, dtype=x.dtype)
    x_cat = jnp.concatenate((x, mask), axis=-1).reshape(-1)
    idx2d = jnp.pad(x_cat.reshape(_NW, b_per_w), ((0, 0), (0, k_pad)))

    out = _gather_call(idx2d, hidden, b_per_w, emb_table)
    return out.reshape(bsz, t, hidden).astype(jnp.float32)


# speedup vs baseline: 8.2126x; 8.2126x over previous
import functools

import jax
import jax.numpy as jnp
from jax import lax
from jax.experimental import pallas as pl
from jax.experimental.pallas import tpu as pltpu
from jax.experimental.pallas import tpu_sc as plsc

MASK_TOKEN_ID = 99999
NATIVE_DRAFT_LEN = 8

_info = plsc.get_sparse_core_info()
_NC = _info.num_cores
_NS = _info.num_subcores
_NW = _NC * _NS


@functools.cache
def _make_body(nw, k, hidden, n_hi, nw_hi, n_lo, rows):
    mesh = plsc.VectorSubcoreMesh(core_axis_name="c", subcore_axis_name="s")

    @functools.partial(
        pl.kernel,
        mesh=mesh,
        out_type=jax.ShapeDtypeStruct((rows, hidden), jnp.float32),
        scratch_types=[
            pltpu.VMEM((k,), jnp.int32),
            pltpu.VMEM((k, hidden), jnp.float32),
            pltpu.SemaphoreType.DMA,
        ],
    )
    def body(idx_hbm, table_hbm, out_hbm, idx_v, rows_v, sem):
        wid = lax.axis_index("s") * _NC + lax.axis_index("c")
        pltpu.sync_copy(idx_hbm.at[wid], idx_v)
        pltpu.async_copy(table_hbm.at[idx_v], rows_v, sem).wait()

        @pl.when(wid < nw_hi)
        def _():
            pltpu.sync_copy(
                rows_v.at[pl.ds(0, n_hi)],
                out_hbm.at[pl.ds(wid * n_hi, n_hi)],
            )

        @pl.when(wid >= nw_hi)
        def _():
            base = nw_hi * n_hi + (wid - nw_hi) * n_lo
            pltpu.sync_copy(
                rows_v.at[pl.ds(0, n_lo)],
                out_hbm.at[pl.ds(base, n_lo)],
            )

    return body


def kernel(x, emb_table):
    bsz, seqlen = x.shape
    vocab, hidden = emb_table.shape
    t = seqlen + NATIVE_DRAFT_LEN - 1
    rows = bsz * t
    assert rows % 8 == 0

    chunks = rows // 8
    c_lo, extra = divmod(chunks, _NW)
    n_hi = (c_lo + 1) * 8
    n_lo = c_lo * 8
    nw_hi = extra
    k = n_hi

    mask = jnp.full((bsz, NATIVE_DRAFT_LEN - 1), MASK_TOKEN_ID, dtype=x.dtype)
    x_cat = jnp.concatenate((x, mask), axis=-1).reshape(-1)

    w = jnp.arange(_NW)
    bases = jnp.where(w < nw_hi, w * n_hi, nw_hi * n_hi + (w - nw_hi) * n_lo)
    pos = bases[:, None] + jnp.arange(k)[None, :]
    idx2d = x_cat[jnp.clip(pos, 0, rows - 1)].astype(jnp.int32)

    body = _make_body(_NW, k, hidden, n_hi, nw_hi, n_lo, rows)
    out = body(idx2d, emb_table)
    return out.reshape(bsz, t, hidden).astype(jnp.float32)
